# baseline (speedup 1.0000x reference)
import functools

import jax
import jax.numpy as jnp
from jax import lax
from jax.experimental import pallas as pl
from jax.experimental.pallas import tpu as pltpu
from jax.experimental.pallas import tpu_sc as plsc

N = 10000
NP = 10240
E = 320000
DIN = 128
H = 256
HH = 128
B = 128
EPS = 1e-5

NC = 2
NS = 16
EK = 128
ECT = 160
NEC = NC * NS * ECT // 2
EP = NEC * EK
HCT = NEC // (NC * NS)
RK = 80
NRC = N // RK
ROWS_PER_TILE = NP // NS

_mesh = plsc.VectorSubcoreMesh(core_axis_name="c", subcore_axis_name="s")


def _zero_vmem(ref, nrows, ncols):
    z = jnp.zeros((16,), jnp.float32)

    def body(i, _):
        for f in range(ncols // 16):
            ref[i, pl.ds(f * 16, 16)] = z
        return 0

    lax.fori_loop(0, nrows, body, 0)


def _fill_vmem(ref, nrows, ncols, value):
    v = jnp.full((16,), value, jnp.float32)

    def body(i, _):
        for f in range(ncols // 16):
            ref[i, pl.ds(f * 16, 16)] = v
        return 0

    lax.fori_loop(0, nrows, body, 0)


@functools.partial(
    pl.kernel,
    out_type=(
        jax.ShapeDtypeStruct((NP, HH), jnp.float32),
        jax.ShapeDtypeStruct((NP, HH), jnp.float32),
        jax.ShapeDtypeStruct((B, HH), jnp.float32),
    ),
    mesh=_mesh,
    scratch_types=[
        pltpu.VMEM((1, EK), jnp.int32),
        pltpu.VMEM((1, EK), jnp.int32),
        pltpu.VMEM((EK, HH), jnp.float32),
        pltpu.VMEM_SHARED((NP, HH), jnp.float32),
        pltpu.VMEM_SHARED((B, HH), jnp.float32),
    ],
)
def _sc_histograms(dst_hbm, batch_hbm, dega_hbm, degb_hbm, cnt_hbm, hidx_v,
                   bidx_v, ones_v, deg_acc, cnt_acc):
    c = lax.axis_index("c")
    s = lax.axis_index("s")

    _zero_vmem(ones_v, EK, HH)
    for k in range(5):
        pltpu.sync_copy(ones_v,
                        deg_acc.at[pl.ds(s * ROWS_PER_TILE + k * EK, EK)])

    @pl.when(c == 1)
    def _():
        @pl.when(s < 8)
        def _():
            pltpu.sync_copy(ones_v.at[pl.ds(0, 16)],
                            cnt_acc.at[pl.ds(s * 16, 16)])

    _fill_vmem(ones_v, EK, HH, 1.0)
    plsc.subcore_barrier()

    def deg_body(j, _):
        chunk = (c * NS + s) * HCT + j
        pltpu.sync_copy(dst_hbm.at[pl.ds(chunk * EK, EK)], hidx_v.at[0])
        pltpu.sync_copy(ones_v, deg_acc.at[hidx_v.at[0]], add=True)
        return 0

    lax.fori_loop(0, HCT, deg_body, 0)

    @pl.when(c == 1)
    def _():
        def cnt_body(j, _):
            chunk = s + NS * j

            @pl.when(chunk < NRC)
            def _():
                pltpu.sync_copy(batch_hbm.at[pl.ds(chunk * RK, RK)],
                                bidx_v.at[0, pl.ds(0, RK)])
                pltpu.sync_copy(ones_v.at[pl.ds(0, RK)],
                                cnt_acc.at[bidx_v.at[0, pl.ds(0, RK)]],
                                add=True)
            return 0

        lax.fori_loop(0, (NRC + NS - 1) // NS, cnt_body, 0)

    plsc.subcore_barrier()

    rows = pl.ds(s * ROWS_PER_TILE, ROWS_PER_TILE)

    @pl.when(c == 0)
    def _():
        pltpu.sync_copy(deg_acc.at[rows], dega_hbm.at[rows])

    @pl.when(c == 1)
    def _():
        pltpu.sync_copy(deg_acc.at[rows], degb_hbm.at[rows])

        @pl.when(s < 8)
        def _():
            pltpu.sync_copy(cnt_acc.at[pl.ds(s * 16, 16)],
                            cnt_hbm.at[pl.ds(s * 16, 16)])


CB = 40
CBW = CB * EK


@functools.partial(
    pl.kernel,
    out_type=(
        jax.ShapeDtypeStruct((NP, HH), jnp.float32),
        jax.ShapeDtypeStruct((NP, HH), jnp.float32),
    ),
    mesh=_mesh,
    scratch_types=[
        pltpu.VMEM((CBW,), jnp.int32),
        pltpu.VMEM((CBW,), jnp.int32),
        pltpu.VMEM((1, EK), jnp.int32),
        pltpu.VMEM((2, EK, HH), jnp.float32),
        pltpu.VMEM_SHARED((NP, HH), jnp.float32),
        pltpu.SemaphoreType.DMA((2,)),
    ],
)
def _sc_edge_scatter(u0_hbm, u1_hbm, src_hbm, dst_hbm, s0_hbm, s1_hbm,
                     sidx1d, didx1d, didx2, rows_v, acc, sems):
    c = lax.axis_index("c")
    s = lax.axis_index("s")

    def zbody(i, _):
        for f in range(HH // 16):
            rows_v[0, i, pl.ds(f * 16, 16)] = jnp.zeros((16,), jnp.float32)
        return 0

    lax.fori_loop(0, EK, zbody, 0)
    for k in range(5):
        pltpu.sync_copy(rows_v.at[0],
                        acc.at[pl.ds(s * ROWS_PER_TILE + k * EK, EK)])
    plsc.subcore_barrier()

    def gather_start(i, b):
        sl = pl.ds(i * EK, EK)

        @pl.when(c == 0)
        def _():
            pltpu.async_copy(u0_hbm.at[sidx1d.at[sl]], rows_v.at[b],
                             sems.at[b])

        @pl.when(c == 1)
        def _():
            pltpu.async_copy(u1_hbm.at[sidx1d.at[sl]], rows_v.at[b],
                             sems.at[b])

    def gather_wait(i, b):
        sl = pl.ds(i * EK, EK)

        @pl.when(c == 0)
        def _():
            pltpu.make_async_copy(u0_hbm.at[sidx1d.at[sl]], rows_v.at[b],
                                  sems.at[b]).wait()

        @pl.when(c == 1)
        def _():
            pltpu.make_async_copy(u1_hbm.at[sidx1d.at[sl]], rows_v.at[b],
                                  sems.at[b]).wait()

    def body(m, _):
        for b in range(2):
            i = m * 2 + b
            gather_wait(i, b)

            @pl.when(i + 2 < CB)
            def _():
                gather_start(i + 2, b)

            for f in range(EK // 16):
                didx2[0, pl.ds(f * 16, 16)] = didx1d[pl.ds(i * EK + f * 16,
                                                           16)]
            pltpu.sync_copy(rows_v.at[b], acc.at[didx2.at[0]], add=True)
        return 0

    for bi in range(ECT // CB):
        base = (s * ECT + bi * CB) * EK
        pltpu.sync_copy(src_hbm.at[pl.ds(base, CBW)], sidx1d)
        pltpu.sync_copy(dst_hbm.at[pl.ds(base, CBW)], didx1d)
        for b in range(2):
            gather_start(b, b)
        lax.fori_loop(0, CB // 2, body, 0)

    plsc.subcore_barrier()

    rows = pl.ds(s * ROWS_PER_TILE, ROWS_PER_TILE)

    @pl.when(c == 0)
    def _():
        pltpu.sync_copy(acc.at[rows], s0_hbm.at[rows])

    @pl.when(c == 1)
    def _():
        pltpu.sync_copy(acc.at[rows], s1_hbm.at[rows])


@functools.partial(
    pl.kernel,
    out_type=(
        jax.ShapeDtypeStruct((B, HH), jnp.float32),
        jax.ShapeDtypeStruct((B, HH), jnp.float32),
        jax.ShapeDtypeStruct((B, HH), jnp.float32),
        jax.ShapeDtypeStruct((B, HH), jnp.float32),
        jax.ShapeDtypeStruct((B, H), jnp.float32),
        jax.ShapeDtypeStruct((B, H), jnp.float32),
    ),
    mesh=_mesh,
    scratch_types=[
        pltpu.VMEM((RK, HH), jnp.float32),
        pltpu.VMEM((RK, HH), jnp.float32),
        pltpu.VMEM((1, RK), jnp.int32),
        pltpu.VMEM((B, H), jnp.float32),
        pltpu.VMEM((8, H), jnp.float32),
        pltpu.VMEM((8, H), jnp.float32),
        pltpu.VMEM_SHARED((B, HH), jnp.float32),
        pltpu.VMEM_SHARED((B, HH), jnp.float32),
        pltpu.VMEM_SHARED((NS, B, H), jnp.float32),
    ],
)
def _sc_pool(ha_hbm, hb_hbm, batch_hbm, s00_hbm, s01_hbm, s10_hbm, s11_hbm,
             max0_hbm, max1_hbm, cv0, cv1, bidx_v, accm_v, red_v, stage_v,
             sum_acc0, sum_acc1, max_slots):
    c = lax.axis_index("c")
    s = lax.axis_index("s")
    wid = c * NS + s

    _zero_vmem(cv0, 8, HH)
    sl8 = pl.ds(s * 8, 8)
    pltpu.sync_copy(cv0.at[pl.ds(0, 8)], sum_acc0.at[sl8])
    pltpu.sync_copy(cv0.at[pl.ds(0, 8)], sum_acc1.at[sl8])
    _fill_vmem(accm_v, B, H, float("-inf"))
    plsc.subcore_barrier()

    def chunk_body(j, _):
        chunk = wid + NC * NS * j

        @pl.when(chunk < NRC)
        def _():
            base = chunk * RK
            pltpu.sync_copy(ha_hbm.at[pl.ds(base, RK)], cv0)
            pltpu.sync_copy(hb_hbm.at[pl.ds(base, RK)], cv1)
            pltpu.sync_copy(batch_hbm.at[pl.ds(base, RK)], bidx_v.at[0])
            pltpu.sync_copy(cv0, sum_acc0.at[bidx_v.at[0]], add=True)
            pltpu.sync_copy(cv1, sum_acc1.at[bidx_v.at[0]], add=True)

            def group_body(q, _):
                bvec = bidx_v[0, pl.ds(q * 16, 16)]
                for r in range(16):
                    g = bvec[r]
                    for f in range(HH // 16):
                        sl = pl.ds(f * 16, 16)
                        accm_v[g, sl] = jnp.maximum(accm_v[g, sl],
                                                    cv0[q * 16 + r, sl])
                        sl2 = pl.ds(HH + f * 16, 16)
                        accm_v[g, sl2] = jnp.maximum(accm_v[g, sl2],
                                                     cv1[q * 16 + r, sl])
                return 0

            lax.fori_loop(0, RK // 16, group_body, 0)
        return 0

    lax.fori_loop(0, (NRC + NC * NS - 1) // (NC * NS), chunk_body, 0)

    pltpu.sync_copy(accm_v, max_slots.at[s])
    plsc.subcore_barrier()

    pltpu.sync_copy(max_slots.at[0, sl8], red_v)

    def merge_body(k, _):
        pltpu.sync_copy(max_slots.at[k + 1, sl8], stage_v)
        for r in range(8):
            for f in range(H // 16):
                sl = pl.ds(f * 16, 16)
                red_v[r, sl] = jnp.maximum(red_v[r, sl], stage_v[r, sl])
        return 0

    lax.fori_loop(0, NS - 1, merge_body, 0)

    @pl.when(c == 0)
    def _():
        pltpu.sync_copy(red_v, max0_hbm.at[sl8])
        pltpu.sync_copy(sum_acc0.at[sl8], s00_hbm.at[sl8])
        pltpu.sync_copy(sum_acc1.at[sl8], s01_hbm.at[sl8])

    @pl.when(c == 1)
    def _():
        pltpu.sync_copy(red_v, max1_hbm.at[sl8])
        pltpu.sync_copy(sum_acc0.at[sl8], s10_hbm.at[sl8])
        pltpu.sync_copy(sum_acc1.at[sl8], s11_hbm.at[sl8])


BN = 1024


def _tc_first_body(dega_ref, degb_ref, x_ref, w_ref, u0_ref, u1_ref):
    dinv = lax.rsqrt(dega_ref[:, 0:1] + degb_ref[:, 0:1] + 1.0)
    hw = jnp.dot(x_ref[...], w_ref[...], preferred_element_type=jnp.float32)
    u0_ref[...] = dinv * hw[:, :HH]
    u1_ref[...] = dinv * hw[:, HH:]


def _tc_first(x, W1, dega, degb):
    return pl.pallas_call(
        _tc_first_body,
        grid=(NP // BN,),
        in_specs=[
            pl.BlockSpec((BN, HH), lambda i: (i, 0)),
            pl.BlockSpec((BN, HH), lambda i: (i, 0)),
            pl.BlockSpec((BN, DIN), lambda i: (i, 0)),
            pl.BlockSpec((DIN, H), lambda i: (0, 0)),
        ],
        out_specs=[
            pl.BlockSpec((BN, HH), lambda i: (i, 0)),
            pl.BlockSpec((BN, HH), lambda i: (i, 0)),
        ],
        out_shape=[
            jax.ShapeDtypeStruct((NP, HH), jnp.float32),
            jax.ShapeDtypeStruct((NP, HH), jnp.float32),
        ],
    )(dega, degb, x, W1)


def _tc_z_body(dega_ref, degb_ref, s0_ref, s1_ref, u0_ref, u1_ref, b_ref,
               z_ref, sums_ref):
    i = pl.program_id(0)
    dinv = lax.rsqrt(dega_ref[:, 0:1] + degb_ref[:, 0:1] + 1.0)
    bias = b_ref[...]
    z0 = dinv * (s0_ref[...] + u0_ref[...]) + bias[None, :HH]
    z1 = dinv * (s1_ref[...] + u1_ref[...]) + bias[None, HH:]
    z = jnp.concatenate([z0, z1], axis=1)
    z_ref[...] = z

    @pl.when(i == 0)
    def _():
        sums_ref[...] = jnp.zeros_like(sums_ref)

    row = i * BN + lax.broadcasted_iota(jnp.int32, (BN, 1), 0)
    zm = jnp.where(row < N, z, 0.0)
    sums_ref[0, :] += jnp.sum(zm, axis=0)
    sums_ref[1, :] += jnp.sum(zm * zm, axis=0)


def _tc_z(s0, s1, u0, u1, bvec, dega, degb):
    return pl.pallas_call(
        _tc_z_body,
        grid=(NP // BN,),
        in_specs=[
            pl.BlockSpec((BN, HH), lambda i: (i, 0)),
            pl.BlockSpec((BN, HH), lambda i: (i, 0)),
            pl.BlockSpec((BN, HH), lambda i: (i, 0)),
            pl.BlockSpec((BN, HH), lambda i: (i, 0)),
            pl.BlockSpec((BN, HH), lambda i: (i, 0)),
            pl.BlockSpec((BN, HH), lambda i: (i, 0)),
            pl.BlockSpec((H,), lambda i: (0,)),
        ],
        out_specs=[
            pl.BlockSpec((BN, H), lambda i: (i, 0)),
            pl.BlockSpec((8, H), lambda i: (0, 0)),
        ],
        out_shape=[
            jax.ShapeDtypeStruct((NP, H), jnp.float32),
            jax.ShapeDtypeStruct((8, H), jnp.float32),
        ],
    )(dega, degb, s0, s1, u0, u1, bvec)


def _bn_affine(sums_ref, g_ref, be_ref):
    mu = sums_ref[0, :] * (1.0 / N)
    msq = sums_ref[1, :] * (1.0 / N)
    var = msq - mu * mu
    a = g_ref[...] * lax.rsqrt(var + EPS)
    return a, be_ref[...] - mu * a


def _tc_mid_body(dega_ref, degb_ref, z_ref, sums_ref, g_ref, be_ref, w_ref,
                 u0_ref, u1_ref):
    a, cb = _bn_affine(sums_ref, g_ref, be_ref)
    h = jnp.maximum(z_ref[...] * a[None, :] + cb[None, :], 0.0)
    hw = jnp.dot(h, w_ref[...], preferred_element_type=jnp.float32)
    dinv = lax.rsqrt(dega_ref[:, 0:1] + degb_ref[:, 0:1] + 1.0)
    u0_ref[...] = dinv * hw[:, :HH]
    u1_ref[...] = dinv * hw[:, HH:]


def _tc_mid(z, sums, g, be, W, dega, degb):
    return pl.pallas_call(
        _tc_mid_body,
        grid=(NP // BN,),
        in_specs=[
            pl.BlockSpec((BN, HH), lambda i: (i, 0)),
            pl.BlockSpec((BN, HH), lambda i: (i, 0)),
            pl.BlockSpec((BN, H), lambda i: (i, 0)),
            pl.BlockSpec((8, H), lambda i: (0, 0)),
            pl.BlockSpec((H,), lambda i: (0,)),
            pl.BlockSpec((H,), lambda i: (0,)),
            pl.BlockSpec((H, H), lambda i: (0, 0)),
        ],
        out_specs=[
            pl.BlockSpec((BN, HH), lambda i: (i, 0)),
            pl.BlockSpec((BN, HH), lambda i: (i, 0)),
        ],
        out_shape=[
            jax.ShapeDtypeStruct((NP, HH), jnp.float32),
            jax.ShapeDtypeStruct((NP, HH), jnp.float32),
        ],
    )(dega, degb, z, sums, g, be, W)


def _tc_h3_body(z_ref, sums_ref, g_ref, be_ref, ha_ref, hb_ref):
    a, cb = _bn_affine(sums_ref, g_ref, be_ref)
    h = jnp.maximum(z_ref[...] * a[None, :] + cb[None, :], 0.0)
    ha_ref[...] = h[:, :HH]
    hb_ref[...] = h[:, HH:]


def _tc_h3(z, sums, g, be):
    return pl.pallas_call(
        _tc_h3_body,
        grid=(NP // BN,),
        in_specs=[
            pl.BlockSpec((BN, H), lambda i: (i, 0)),
            pl.BlockSpec((8, H), lambda i: (0, 0)),
            pl.BlockSpec((H,), lambda i: (0,)),
            pl.BlockSpec((H,), lambda i: (0,)),
        ],
        out_specs=[
            pl.BlockSpec((BN, HH), lambda i: (i, 0)),
            pl.BlockSpec((BN, HH), lambda i: (i, 0)),
        ],
        out_shape=[
            jax.ShapeDtypeStruct((NP, HH), jnp.float32),
            jax.ShapeDtypeStruct((NP, HH), jnp.float32),
        ],
    )(z, sums, g, be)


def _tc_combine_body(s00_ref, s01_ref, s10_ref, s11_ref, max0_ref, max1_ref,
                     cnt_ref, out_ref):
    cnt = cnt_ref[:, 0:1]
    denom = jnp.maximum(cnt, 1.0)
    out_ref[:, :HH] = (s00_ref[...] + s10_ref[...]) / denom
    out_ref[:, HH:H] = (s01_ref[...] + s11_ref[...]) / denom
    mx = jnp.maximum(max0_ref[...], max1_ref[...])
    out_ref[:, H:] = jnp.where(cnt > 0.0, mx, 0.0)


def _tc_combine(s00, s01, s10, s11, max0, max1, cnt):
    return pl.pallas_call(
        _tc_combine_body,
        grid=(1,),
        in_specs=[
            pl.BlockSpec((B, HH), lambda i: (0, 0)),
            pl.BlockSpec((B, HH), lambda i: (0, 0)),
            pl.BlockSpec((B, HH), lambda i: (0, 0)),
            pl.BlockSpec((B, HH), lambda i: (0, 0)),
            pl.BlockSpec((B, H), lambda i: (0, 0)),
            pl.BlockSpec((B, H), lambda i: (0, 0)),
            pl.BlockSpec((B, HH), lambda i: (0, 0)),
        ],
        out_specs=pl.BlockSpec((B, 2 * H), lambda i: (0, 0)),
        out_shape=jax.ShapeDtypeStruct((B, 2 * H), jnp.float32),
    )(s00, s01, s10, s11, max0, max1, cnt)


def kernel(x, edge_index, batch, W1, b1, g1, be1, W2, b2, g2, be2,
           W3, b3, g3, be3):
    src = jnp.pad(edge_index[0], (0, EP - E), constant_values=N)
    dst = jnp.pad(edge_index[1], (0, EP - E), constant_values=N)
    x = jnp.pad(x, ((0, NP - N), (0, 0)))

    dega, degb, cnt = _sc_histograms(dst, batch)

    u0, u1 = _tc_first(x, W1, dega, degb)
    s0, s1 = _sc_edge_scatter(u0, u1, src, dst)
    z, sums = _tc_z(s0, s1, u0, u1, b1, dega, degb)

    u0, u1 = _tc_mid(z, sums, g1, be1, W2, dega, degb)
    s0, s1 = _sc_edge_scatter(u0, u1, src, dst)
    z, sums = _tc_z(s0, s1, u0, u1, b2, dega, degb)

    u0, u1 = _tc_mid(z, sums, g2, be2, W3, dega, degb)
    s0, s1 = _sc_edge_scatter(u0, u1, src, dst)
    z, sums = _tc_z(s0, s1, u0, u1, b3, dega, degb)

    ha, hb = _tc_h3(z, sums, g3, be3)
    s00, s01, s10, s11, max0, max1 = _sc_pool(ha, hb, batch)
    return _tc_combine(s00, s01, s10, s11, max0, max1, cnt)


# speedup vs baseline: 2.0879x; 2.0879x over previous
import functools

import jax
import jax.numpy as jnp
from jax import lax
from jax.experimental import pallas as pl
from jax.experimental.pallas import tpu as pltpu
from jax.experimental.pallas import tpu_sc as plsc

N = 10000
NP = 10240
E = 320000
DIN = 128
H = 256
HH = 128
B = 128
EPS = 1e-5

NC = 2
NS = 16
EK = 128
ECT = 160
NEC = NC * NS * ECT // 2
EP = NEC * EK
HCT = NEC // (NC * NS)
NECR = E // EK
RK = 80
NRC = N // RK
ROWS_PER_TILE = NP // NS

_mesh = plsc.VectorSubcoreMesh(core_axis_name="c", subcore_axis_name="s")


def _zero_vmem(ref, nrows, ncols):
    z = jnp.zeros((16,), jnp.float32)

    def body(i, _):
        for f in range(ncols // 16):
            ref[i, pl.ds(f * 16, 16)] = z
        return 0

    lax.fori_loop(0, nrows, body, 0)


def _fill_vmem(ref, nrows, ncols, value):
    v = jnp.full((16,), value, jnp.float32)

    def body(i, _):
        for f in range(ncols // 16):
            ref[i, pl.ds(f * 16, 16)] = v
        return 0

    lax.fori_loop(0, nrows, body, 0)


@functools.partial(
    pl.kernel,
    out_type=(
        jax.ShapeDtypeStruct((NP, HH), jnp.float32),
        jax.ShapeDtypeStruct((NP, HH), jnp.float32),
        jax.ShapeDtypeStruct((B, HH), jnp.float32),
    ),
    mesh=_mesh,
    scratch_types=[
        pltpu.VMEM((1, EK), jnp.int32),
        pltpu.VMEM((1, EK), jnp.int32),
        pltpu.VMEM((EK, HH), jnp.float32),
        pltpu.VMEM_SHARED((NP, HH), jnp.float32),
        pltpu.VMEM_SHARED((B, HH), jnp.float32),
    ],
)
def _sc_histograms(dst_hbm, batch_hbm, dega_hbm, degb_hbm, cnt_hbm, hidx_v,
                   bidx_v, ones_v, deg_acc, cnt_acc):
    c = lax.axis_index("c")
    s = lax.axis_index("s")

    _zero_vmem(ones_v, EK, HH)
    for k in range(5):
        pltpu.sync_copy(ones_v,
                        deg_acc.at[pl.ds(s * ROWS_PER_TILE + k * EK, EK)])

    @pl.when(c == 1)
    def _():
        @pl.when(s < 8)
        def _():
            pltpu.sync_copy(ones_v.at[pl.ds(0, 16)],
                            cnt_acc.at[pl.ds(s * 16, 16)])

    _fill_vmem(ones_v, EK, HH, 1.0)
    plsc.subcore_barrier()

    def deg_body(j, _):
        chunk = (c * NS + s) * HCT + j

        @pl.when(chunk < NECR)
        def _():
            pltpu.sync_copy(dst_hbm.at[pl.ds(chunk * EK, EK)], hidx_v.at[0])
            pltpu.sync_copy(ones_v, deg_acc.at[hidx_v.at[0]], add=True)
        return 0

    lax.fori_loop(0, HCT, deg_body, 0)

    @pl.when(c == 1)
    def _():
        def cnt_body(j, _):
            chunk = s + NS * j

            @pl.when(chunk < NRC)
            def _():
                pltpu.sync_copy(batch_hbm.at[pl.ds(chunk * RK, RK)],
                                bidx_v.at[0, pl.ds(0, RK)])
                pltpu.sync_copy(ones_v.at[pl.ds(0, RK)],
                                cnt_acc.at[bidx_v.at[0, pl.ds(0, RK)]],
                                add=True)
            return 0

        lax.fori_loop(0, (NRC + NS - 1) // NS, cnt_body, 0)

    plsc.subcore_barrier()

    rows = pl.ds(s * ROWS_PER_TILE, ROWS_PER_TILE)

    @pl.when(c == 0)
    def _():
        pltpu.sync_copy(deg_acc.at[rows], dega_hbm.at[rows])

    @pl.when(c == 1)
    def _():
        pltpu.sync_copy(deg_acc.at[rows], degb_hbm.at[rows])

        @pl.when(s < 8)
        def _():
            pltpu.sync_copy(cnt_acc.at[pl.ds(s * 16, 16)],
                            cnt_hbm.at[pl.ds(s * 16, 16)])


CB = 40
CBW = CB * EK


@functools.partial(
    pl.kernel,
    out_type=(
        jax.ShapeDtypeStruct((NP, HH), jnp.float32),
        jax.ShapeDtypeStruct((NP, HH), jnp.float32),
    ),
    mesh=_mesh,
    scratch_types=[
        pltpu.VMEM((CBW,), jnp.int32),
        pltpu.VMEM((CBW,), jnp.int32),
        pltpu.VMEM((1, EK), jnp.int32),
        pltpu.VMEM((2, EK, HH), jnp.float32),
        pltpu.VMEM_SHARED((NP, HH), jnp.float32),
        pltpu.SemaphoreType.DMA((2,)),
    ],
)
def _sc_edge_scatter(u0_hbm, u1_hbm, src_hbm, dst_hbm, s0_hbm, s1_hbm,
                     sidx1d, didx1d, didx2, rows_v, acc, sems):
    c = lax.axis_index("c")
    s = lax.axis_index("s")

    def zbody(i, _):
        for f in range(HH // 16):
            rows_v[0, i, pl.ds(f * 16, 16)] = jnp.zeros((16,), jnp.float32)
        return 0

    lax.fori_loop(0, EK, zbody, 0)
    for k in range(5):
        pltpu.sync_copy(rows_v.at[0],
                        acc.at[pl.ds(s * ROWS_PER_TILE + k * EK, EK)])
    plsc.subcore_barrier()

    def gather_start(i, b):
        sl = pl.ds(i * EK, EK)

        @pl.when(c == 0)
        def _():
            pltpu.async_copy(u0_hbm.at[sidx1d.at[sl]], rows_v.at[b],
                             sems.at[b])

        @pl.when(c == 1)
        def _():
            pltpu.async_copy(u1_hbm.at[sidx1d.at[sl]], rows_v.at[b],
                             sems.at[b])

    def gather_wait(i, b):
        sl = pl.ds(i * EK, EK)

        @pl.when(c == 0)
        def _():
            pltpu.make_async_copy(u0_hbm.at[sidx1d.at[sl]], rows_v.at[b],
                                  sems.at[b]).wait()

        @pl.when(c == 1)
        def _():
            pltpu.make_async_copy(u1_hbm.at[sidx1d.at[sl]], rows_v.at[b],
                                  sems.at[b]).wait()

    for bi in range(ECT // CB):
        gbase = s * ECT + bi * CB

        def body(m, _, gbase=gbase):
            for b in range(2):
                i = m * 2 + b

                @pl.when(gbase + i < NECR)
                def _(i=i, b=b):
                    gather_wait(i, b)

                    @pl.when((i + 2 < CB) & (gbase + i + 2 < NECR))
                    def _():
                        gather_start(i + 2, b)

                    for f in range(EK // 16):
                        didx2[0, pl.ds(f * 16, 16)] = didx1d[
                            pl.ds(i * EK + f * 16, 16)]
                    pltpu.sync_copy(rows_v.at[b], acc.at[didx2.at[0]],
                                    add=True)
            return 0

        pltpu.sync_copy(src_hbm.at[pl.ds(gbase * EK, CBW)], sidx1d)
        pltpu.sync_copy(dst_hbm.at[pl.ds(gbase * EK, CBW)], didx1d)
        for b in range(2):
            @pl.when(gbase + b < NECR)
            def _(b=b):
                gather_start(b, b)
        lax.fori_loop(0, CB // 2, body, 0)

    plsc.subcore_barrier()

    rows = pl.ds(s * ROWS_PER_TILE, ROWS_PER_TILE)

    @pl.when(c == 0)
    def _():
        pltpu.sync_copy(acc.at[rows], s0_hbm.at[rows])

    @pl.when(c == 1)
    def _():
        pltpu.sync_copy(acc.at[rows], s1_hbm.at[rows])


@functools.partial(
    pl.kernel,
    out_type=(
        jax.ShapeDtypeStruct((B, HH), jnp.float32),
        jax.ShapeDtypeStruct((B, HH), jnp.float32),
        jax.ShapeDtypeStruct((B, HH), jnp.float32),
        jax.ShapeDtypeStruct((B, HH), jnp.float32),
        jax.ShapeDtypeStruct((B, H), jnp.float32),
        jax.ShapeDtypeStruct((B, H), jnp.float32),
    ),
    mesh=_mesh,
    scratch_types=[
        pltpu.VMEM((RK, HH), jnp.float32),
        pltpu.VMEM((RK, HH), jnp.float32),
        pltpu.VMEM((1, RK), jnp.int32),
        pltpu.VMEM((B, H), jnp.float32),
        pltpu.VMEM((8, H), jnp.float32),
        pltpu.VMEM((8, H), jnp.float32),
        pltpu.VMEM_SHARED((B, HH), jnp.float32),
        pltpu.VMEM_SHARED((B, HH), jnp.float32),
        pltpu.VMEM_SHARED((NS, B, H), jnp.float32),
    ],
)
def _sc_pool(ha_hbm, hb_hbm, batch_hbm, s00_hbm, s01_hbm, s10_hbm, s11_hbm,
             max0_hbm, max1_hbm, cv0, cv1, bidx_v, accm_v, red_v, stage_v,
             sum_acc0, sum_acc1, max_slots):
    c = lax.axis_index("c")
    s = lax.axis_index("s")
    wid = c * NS + s

    _zero_vmem(cv0, 8, HH)
    sl8 = pl.ds(s * 8, 8)
    pltpu.sync_copy(cv0.at[pl.ds(0, 8)], sum_acc0.at[sl8])
    pltpu.sync_copy(cv0.at[pl.ds(0, 8)], sum_acc1.at[sl8])
    _fill_vmem(accm_v, B, H, float("-inf"))
    plsc.subcore_barrier()

    def chunk_body(j, _):
        chunk = wid + NC * NS * j

        @pl.when(chunk < NRC)
        def _():
            base = chunk * RK
            pltpu.sync_copy(ha_hbm.at[pl.ds(base, RK)], cv0)
            pltpu.sync_copy(hb_hbm.at[pl.ds(base, RK)], cv1)
            pltpu.sync_copy(batch_hbm.at[pl.ds(base, RK)], bidx_v.at[0])
            pltpu.sync_copy(cv0, sum_acc0.at[bidx_v.at[0]], add=True)
            pltpu.sync_copy(cv1, sum_acc1.at[bidx_v.at[0]], add=True)

            def group_body(q, _):
                bvec = bidx_v[0, pl.ds(q * 16, 16)]
                for r in range(16):
                    g = bvec[r]
                    for f in range(HH // 16):
                        sl = pl.ds(f * 16, 16)
                        accm_v[g, sl] = jnp.maximum(accm_v[g, sl],
                                                    cv0[q * 16 + r, sl])
                        sl2 = pl.ds(HH + f * 16, 16)
                        accm_v[g, sl2] = jnp.maximum(accm_v[g, sl2],
                                                     cv1[q * 16 + r, sl])
                return 0

            lax.fori_loop(0, RK // 16, group_body, 0)
        return 0

    lax.fori_loop(0, (NRC + NC * NS - 1) // (NC * NS), chunk_body, 0)

    pltpu.sync_copy(accm_v, max_slots.at[s])
    plsc.subcore_barrier()

    pltpu.sync_copy(max_slots.at[0, sl8], red_v)

    def merge_body(k, _):
        pltpu.sync_copy(max_slots.at[k + 1, sl8], stage_v)
        for r in range(8):
            for f in range(H // 16):
                sl = pl.ds(f * 16, 16)
                red_v[r, sl] = jnp.maximum(red_v[r, sl], stage_v[r, sl])
        return 0

    lax.fori_loop(0, NS - 1, merge_body, 0)

    @pl.when(c == 0)
    def _():
        pltpu.sync_copy(red_v, max0_hbm.at[sl8])
        pltpu.sync_copy(sum_acc0.at[sl8], s00_hbm.at[sl8])
        pltpu.sync_copy(sum_acc1.at[sl8], s01_hbm.at[sl8])

    @pl.when(c == 1)
    def _():
        pltpu.sync_copy(red_v, max1_hbm.at[sl8])
        pltpu.sync_copy(sum_acc0.at[sl8], s10_hbm.at[sl8])
        pltpu.sync_copy(sum_acc1.at[sl8], s11_hbm.at[sl8])


BN = 1024


def _tc_first_body(dega_ref, degb_ref, x_ref, w_ref, u0_ref, u1_ref):
    dinv = lax.rsqrt(dega_ref[:, 0:1] + degb_ref[:, 0:1] + 1.0)
    hw = jnp.dot(x_ref[...], w_ref[...], preferred_element_type=jnp.float32)
    u0_ref[...] = dinv * hw[:, :HH]
    u1_ref[...] = dinv * hw[:, HH:]


def _tc_first(x, W1, dega, degb):
    return pl.pallas_call(
        _tc_first_body,
        grid=(NP // BN,),
        in_specs=[
            pl.BlockSpec((BN, HH), lambda i: (i, 0)),
            pl.BlockSpec((BN, HH), lambda i: (i, 0)),
            pl.BlockSpec((BN, DIN), lambda i: (i, 0)),
            pl.BlockSpec((DIN, H), lambda i: (0, 0)),
        ],
        out_specs=[
            pl.BlockSpec((BN, HH), lambda i: (i, 0)),
            pl.BlockSpec((BN, HH), lambda i: (i, 0)),
        ],
        out_shape=[
            jax.ShapeDtypeStruct((NP, HH), jnp.float32),
            jax.ShapeDtypeStruct((NP, HH), jnp.float32),
        ],
    )(dega, degb, x, W1)


def _tc_z_body(dega_ref, degb_ref, s0_ref, s1_ref, u0_ref, u1_ref, b_ref,
               z_ref, sums_ref):
    i = pl.program_id(0)
    dinv = lax.rsqrt(dega_ref[:, 0:1] + degb_ref[:, 0:1] + 1.0)
    bias = b_ref[...]
    z0 = dinv * (s0_ref[...] + u0_ref[...]) + bias[None, :HH]
    z1 = dinv * (s1_ref[...] + u1_ref[...]) + bias[None, HH:]
    z = jnp.concatenate([z0, z1], axis=1)
    z_ref[...] = z

    @pl.when(i == 0)
    def _():
        sums_ref[...] = jnp.zeros_like(sums_ref)

    row = i * BN + lax.broadcasted_iota(jnp.int32, (BN, 1), 0)
    zm = jnp.where(row < N, z, 0.0)
    sums_ref[0, :] += jnp.sum(zm, axis=0)
    sums_ref[1, :] += jnp.sum(zm * zm, axis=0)


def _tc_z(s0, s1, u0, u1, bvec, dega, degb):
    return pl.pallas_call(
        _tc_z_body,
        grid=(NP // BN,),
        in_specs=[
            pl.BlockSpec((BN, HH), lambda i: (i, 0)),
            pl.BlockSpec((BN, HH), lambda i: (i, 0)),
            pl.BlockSpec((BN, HH), lambda i: (i, 0)),
            pl.BlockSpec((BN, HH), lambda i: (i, 0)),
            pl.BlockSpec((BN, HH), lambda i: (i, 0)),
            pl.BlockSpec((BN, HH), lambda i: (i, 0)),
            pl.BlockSpec((H,), lambda i: (0,)),
        ],
        out_specs=[
            pl.BlockSpec((BN, H), lambda i: (i, 0)),
            pl.BlockSpec((8, H), lambda i: (0, 0)),
        ],
        out_shape=[
            jax.ShapeDtypeStruct((NP, H), jnp.float32),
            jax.ShapeDtypeStruct((8, H), jnp.float32),
        ],
    )(dega, degb, s0, s1, u0, u1, bvec)


def _bn_affine(sums_ref, g_ref, be_ref):
    mu = sums_ref[0, :] * (1.0 / N)
    msq = sums_ref[1, :] * (1.0 / N)
    var = msq - mu * mu
    a = g_ref[...] * lax.rsqrt(var + EPS)
    return a, be_ref[...] - mu * a


def _tc_mid_body(dega_ref, degb_ref, z_ref, sums_ref, g_ref, be_ref, w_ref,
                 u0_ref, u1_ref):
    a, cb = _bn_affine(sums_ref, g_ref, be_ref)
    h = jnp.maximum(z_ref[...] * a[None, :] + cb[None, :], 0.0)
    hw = jnp.dot(h, w_ref[...], preferred_element_type=jnp.float32)
    dinv = lax.rsqrt(dega_ref[:, 0:1] + degb_ref[:, 0:1] + 1.0)
    u0_ref[...] = dinv * hw[:, :HH]
    u1_ref[...] = dinv * hw[:, HH:]


def _tc_mid(z, sums, g, be, W, dega, degb):
    return pl.pallas_call(
        _tc_mid_body,
        grid=(NP // BN,),
        in_specs=[
            pl.BlockSpec((BN, HH), lambda i: (i, 0)),
            pl.BlockSpec((BN, HH), lambda i: (i, 0)),
            pl.BlockSpec((BN, H), lambda i: (i, 0)),
            pl.BlockSpec((8, H), lambda i: (0, 0)),
            pl.BlockSpec((H,), lambda i: (0,)),
            pl.BlockSpec((H,), lambda i: (0,)),
            pl.BlockSpec((H, H), lambda i: (0, 0)),
        ],
        out_specs=[
            pl.BlockSpec((BN, HH), lambda i: (i, 0)),
            pl.BlockSpec((BN, HH), lambda i: (i, 0)),
        ],
        out_shape=[
            jax.ShapeDtypeStruct((NP, HH), jnp.float32),
            jax.ShapeDtypeStruct((NP, HH), jnp.float32),
        ],
    )(dega, degb, z, sums, g, be, W)


def _tc_h3_body(z_ref, sums_ref, g_ref, be_ref, ha_ref, hb_ref):
    a, cb = _bn_affine(sums_ref, g_ref, be_ref)
    h = jnp.maximum(z_ref[...] * a[None, :] + cb[None, :], 0.0)
    ha_ref[...] = h[:, :HH]
    hb_ref[...] = h[:, HH:]


def _tc_h3(z, sums, g, be):
    return pl.pallas_call(
        _tc_h3_body,
        grid=(NP // BN,),
        in_specs=[
            pl.BlockSpec((BN, H), lambda i: (i, 0)),
            pl.BlockSpec((8, H), lambda i: (0, 0)),
            pl.BlockSpec((H,), lambda i: (0,)),
            pl.BlockSpec((H,), lambda i: (0,)),
        ],
        out_specs=[
            pl.BlockSpec((BN, HH), lambda i: (i, 0)),
            pl.BlockSpec((BN, HH), lambda i: (i, 0)),
        ],
        out_shape=[
            jax.ShapeDtypeStruct((NP, HH), jnp.float32),
            jax.ShapeDtypeStruct((NP, HH), jnp.float32),
        ],
    )(z, sums, g, be)


def _tc_combine_body(s00_ref, s01_ref, s10_ref, s11_ref, max0_ref, max1_ref,
                     cnt_ref, out_ref):
    cnt = cnt_ref[:, 0:1]
    denom = jnp.maximum(cnt, 1.0)
    out_ref[:, :HH] = (s00_ref[...] + s10_ref[...]) / denom
    out_ref[:, HH:H] = (s01_ref[...] + s11_ref[...]) / denom
    mx = jnp.maximum(max0_ref[...], max1_ref[...])
    out_ref[:, H:] = jnp.where(cnt > 0.0, mx, 0.0)


def _tc_combine(s00, s01, s10, s11, max0, max1, cnt):
    return pl.pallas_call(
        _tc_combine_body,
        grid=(1,),
        in_specs=[
            pl.BlockSpec((B, HH), lambda i: (0, 0)),
            pl.BlockSpec((B, HH), lambda i: (0, 0)),
            pl.BlockSpec((B, HH), lambda i: (0, 0)),
            pl.BlockSpec((B, HH), lambda i: (0, 0)),
            pl.BlockSpec((B, H), lambda i: (0, 0)),
            pl.BlockSpec((B, H), lambda i: (0, 0)),
            pl.BlockSpec((B, HH), lambda i: (0, 0)),
        ],
        out_specs=pl.BlockSpec((B, 2 * H), lambda i: (0, 0)),
        out_shape=jax.ShapeDtypeStruct((B, 2 * H), jnp.float32),
    )(s00, s01, s10, s11, max0, max1, cnt)


def kernel(x, edge_index, batch, W1, b1, g1, be1, W2, b2, g2, be2,
           W3, b3, g3, be3):
    src = jnp.pad(edge_index[0], (0, EP - E), constant_values=N)
    dst = jnp.pad(edge_index[1], (0, EP - E), constant_values=N)
    x = jnp.pad(x, ((0, NP - N), (0, 0)))

    dega, degb, cnt = _sc_histograms(dst, batch)

    u0, u1 = _tc_first(x, W1, dega, degb)
    s0, s1 = _sc_edge_scatter(u0, u1, src, dst)
    z, sums = _tc_z(s0, s1, u0, u1, b1, dega, degb)

    u0, u1 = _tc_mid(z, sums, g1, be1, W2, dega, degb)
    s0, s1 = _sc_edge_scatter(u0, u1, src, dst)
    z, sums = _tc_z(s0, s1, u0, u1, b2, dega, degb)

    u0, u1 = _tc_mid(z, sums, g2, be2, W3, dega, degb)
    s0, s1 = _sc_edge_scatter(u0, u1, src, dst)
    z, sums = _tc_z(s0, s1, u0, u1, b3, dega, degb)

    ha, hb = _tc_h3(z, sums, g3, be3)
    s00, s01, s10, s11, max0, max1 = _sc_pool(ha, hb, batch)
    return _tc_combine(s00, s01, s10, s11, max0, max1, cnt)
